# baseline (speedup 1.0000x reference)
import functools

import jax
import jax.numpy as jnp
from jax import lax
from jax.experimental import pallas as pl
from jax.experimental.pallas import tpu as pltpu
from jax.experimental.pallas import tpu_sc as plsc

_N_USERS = 25000
_N_ITEMS = 25000
_N = _N_USERS + _N_ITEMS
_D = 64
_L = 3
_E = 800000

_NC = 2
_NS = 16
_B = 128
_GRP = 2
_BLOCKS_PER_SUB = 392
_EDGES_PER_SUB = _B * _BLOCKS_PER_SUB
_E_PAD = _EDGES_PER_SUB * _NS
_NB = _E_PAD // _B
_GROUPS = _BLOCKS_PER_SUB // _GRP
_N_PAD = 50048
_ROWS_PER_SUB = _N_PAD // _NS


def _spmm_kernel(feat2_h, meta_h, out_lo, out_hi,
                 acc, meta_v, rowbuf, gsem, ssem, isem):
    cid = lax.axis_index("c")
    sid = lax.axis_index("s")
    iota = lax.iota(jnp.int32, 16)
    zeros = jnp.zeros((16,), jnp.float32)
    _GB = _GRP * _B

    for i in range(_GB):
        rowbuf[0, i, pl.ds(0, 16)] = zeros
        rowbuf[0, i, pl.ds(16, 16)] = zeros
    r0 = sid * _ROWS_PER_SUB
    for t in range(_ROWS_PER_SUB // _GB):
        pltpu.sync_copy(rowbuf.at[0], acc.at[pl.ds(r0 + t * _GB, _GB)])
    tail = _ROWS_PER_SUB % _GB
    if tail:
        pltpu.sync_copy(rowbuf.at[0, pl.ds(0, tail)],
                        acc.at[pl.ds(r0 + _ROWS_PER_SUB - tail, tail)])
    plsc.subcore_barrier()


    def _load_meta(g):
        blk = sid * _BLOCKS_PER_SUB + g * _GRP
        ms = g % 4
        pltpu.async_copy(meta_h.at[pl.ds(blk, _GRP)], meta_v.at[ms],
                         isem.at[ms])

    def _fire_gathers(slot, g):
        ms = g % 4
        pltpu.make_async_copy(meta_h.at[pl.ds(0, _GRP)], meta_v.at[ms],
                              isem.at[ms]).wait()
        for jj in range(_GRP):
            pltpu.async_copy(feat2_h.at[meta_v.at[ms, jj, cid + 1]],
                             rowbuf.at[slot, pl.ds(jj * _B, _B)],
                             gsem.at[slot])

    _load_meta(0)
    _fire_gathers(0, 0)
    _load_meta(1)

    def group_body(g, _):
        b = g % 2
        ms = g % 4
        pltpu.make_async_copy(feat2_h.at[pl.ds(0, _GB)], rowbuf.at[b],
                              gsem.at[b]).wait()

        @pl.when(g >= 1)
        def _():
            pltpu.make_async_copy(feat2_h.at[pl.ds(0, _GB)],
                                  rowbuf.at[1 - b], ssem.at[1 - b]).wait()

        @pl.when(g < _GROUPS - 1)
        def _():
            _fire_gathers(1 - b, g + 1)

        @pl.when(g < _GROUPS - 2)
        def _():
            _load_meta(g + 2)

        for jj in range(_GRP):
            @plsc.parallel_loop(0, _B // 16, unroll=2)
            def _(e16):
                vv = plsc.bitcast(
                    meta_v[ms, jj, 3, pl.ds(e16 * 16, 16)], jnp.float32)
                rows16 = iota + (e16 * 16 + jj * _B)
                for d in range(32):
                    sd = jnp.full((16,), d, jnp.int32)
                    x = plsc.load_gather(rowbuf.at[b], [rows16, sd])
                    plsc.store_scatter(rowbuf.at[b], [rows16, sd], x * vv)

        for jj in range(_GRP):
            pltpu.async_copy(rowbuf.at[b, pl.ds(jj * _B, _B)],
                             acc.at[meta_v.at[ms, jj, 0]], ssem.at[b],
                             add=True)
        return 0

    lax.fori_loop(0, _GROUPS, group_body, 0)
    last = (_GROUPS - 1) % 2
    pltpu.make_async_copy(feat2_h.at[pl.ds(0, _GB)], rowbuf.at[last],
                          ssem.at[last]).wait()
    plsc.subcore_barrier()

    @pl.when(cid == 0)
    def _():
        pltpu.sync_copy(acc.at[pl.ds(r0, _ROWS_PER_SUB)],
                        out_lo.at[pl.ds(r0, _ROWS_PER_SUB)])

    @pl.when(cid == 1)
    def _():
        pltpu.sync_copy(acc.at[pl.ds(r0, _ROWS_PER_SUB)],
                        out_hi.at[pl.ds(r0, _ROWS_PER_SUB)])


@jax.jit
def _spmm(feat2, meta):
    mesh = plsc.VectorSubcoreMesh(core_axis_name="c", subcore_axis_name="s")
    f = pl.kernel(
        _spmm_kernel,
        out_type=(jax.ShapeDtypeStruct((_N_PAD, 32), jnp.float32),
                  jax.ShapeDtypeStruct((_N_PAD, 32), jnp.float32)),
        mesh=mesh,
        compiler_params=pltpu.CompilerParams(needs_layout_passes=False,
                                             use_tc_tiling_on_sc=False),
        scratch_types=[
            pltpu.VMEM_SHARED((_N_PAD, 32), jnp.float32),
            pltpu.VMEM((4, _GRP, 4, _B), jnp.int32),
            pltpu.VMEM((2, _GRP * _B, 32), jnp.float32),
            pltpu.SemaphoreType.DMA((2,)),
            pltpu.SemaphoreType.DMA((2,)),
            pltpu.SemaphoreType.DMA((4,)),
        ],
    )
    return f(feat2, meta)


def _dense_body(f_ref, t1lo_ref, t1hi_ref, w_ref, b_ref, o_ref):
    f = f_ref[...]
    t1 = jnp.concatenate([t1lo_ref[...], t1hi_ref[...]], axis=1)
    t2 = t1 * f
    dn = (((1,), (1,)), ((), ()))
    z = (lax.dot_general(t1, w_ref[0], dn, precision=lax.Precision.DEFAULT,
                         preferred_element_type=jnp.float32)
         + lax.dot_general(t2, w_ref[1], dn, precision=lax.Precision.DEFAULT,
                           preferred_element_type=jnp.float32)
         + lax.dot_general(f, w_ref[2], dn, precision=lax.Precision.DEFAULT,
                           preferred_element_type=jnp.float32)
         + b_ref[...])
    z = jnp.where(z >= 0, z, 0.01 * z)
    s = jnp.sum(z * z, axis=1, keepdims=True)
    o_ref[...] = z * lax.rsqrt(jnp.maximum(s, 1e-24))


def _dense(feature, t1lo, t1hi, W, bsum):
    BN = 2000
    nblk = _N // BN
    return pl.pallas_call(
        _dense_body,
        grid=(nblk,),
        in_specs=[
            pl.BlockSpec((BN, _D), lambda i: (i, 0)),
            pl.BlockSpec((BN, 32), lambda i: (i, 0)),
            pl.BlockSpec((BN, 32), lambda i: (i, 0)),
            pl.BlockSpec((3, _D, _D), lambda i: (0, 0, 0)),
            pl.BlockSpec((1, _D), lambda i: (0, 0)),
        ],
        out_specs=pl.BlockSpec((BN, _D), lambda i: (i, 0)),
        out_shape=jax.ShapeDtypeStruct((_N, _D), jnp.float32),
    )(feature, t1lo, t1hi, W, bsum)


def kernel(l_indices, l_values, user_emb, item_emb, Ws, bs):
    rows = l_indices[0].astype(jnp.int32)
    cols = l_indices[1].astype(jnp.int32)
    pad = _E_PAD - _E
    spread = (jnp.arange(pad, dtype=jnp.int32) * 17) % _N
    rows_p = jnp.concatenate([rows, spread])
    cols_p = jnp.concatenate([cols, spread])
    vals_p = jnp.concatenate([l_values, jnp.zeros((pad,), jnp.float32)])
    meta = jnp.stack([
        rows_p.reshape(_NB, _B),
        (cols_p * 2).reshape(_NB, _B),
        (cols_p * 2 + 1).reshape(_NB, _B),
        lax.bitcast_convert_type(vals_p, jnp.int32).reshape(_NB, _B),
    ], axis=1)

    feature = jnp.concatenate([user_emb, item_emb], axis=0)
    all_embs = [feature]
    for i in range(_L):
        t1lo, t1hi = _spmm(feature.reshape(2 * _N, 32), meta)
        feature = _dense(feature, t1lo, t1hi, Ws[i], bs[i].sum(0)[None, :])
        all_embs.append(feature)
    all_e = jnp.concatenate(all_embs, axis=1)
    return all_e[:_N_USERS], all_e[_N_USERS:]


# speedup vs baseline: 3.8780x; 3.8780x over previous
import functools

import jax
import jax.numpy as jnp
from jax import lax
from jax.experimental import pallas as pl
from jax.experimental.pallas import tpu as pltpu
from jax.experimental.pallas import tpu_sc as plsc

_N_USERS = 25000
_N_ITEMS = 25000
_N = _N_USERS + _N_ITEMS
_D = 64
_L = 3
_E = 800000

_NC = 2
_NS = 16
_B = 128
_GRP = 2
_BLOCKS_PER_SUB = 392
_EDGES_PER_SUB = _B * _BLOCKS_PER_SUB
_E_PAD = _EDGES_PER_SUB * _NS
_NB = _E_PAD // _B
_GROUPS = _BLOCKS_PER_SUB // _GRP
_N_PAD = 50048
_ROWS_PER_SUB = _N_PAD // _NS


def _spmm_kernel(feat2_h, meta_h, out_lo, out_hi,
                 acc, meta_v, rowbuf, gsem, ssem, isem):
    cid = lax.axis_index("c")
    sid = lax.axis_index("s")
    zeros = jnp.zeros((16,), jnp.float32)
    _GB = _GRP * _B

    for i in range(_GB):
        rowbuf[0, i, pl.ds(0, 16)] = zeros
        rowbuf[0, i, pl.ds(16, 16)] = zeros
    r0 = sid * _ROWS_PER_SUB
    for t in range(_ROWS_PER_SUB // _GB):
        pltpu.sync_copy(rowbuf.at[0], acc.at[pl.ds(r0 + t * _GB, _GB)])
    tail = _ROWS_PER_SUB % _GB
    if tail:
        pltpu.sync_copy(rowbuf.at[0, pl.ds(0, tail)],
                        acc.at[pl.ds(r0 + _ROWS_PER_SUB - tail, tail)])
    plsc.subcore_barrier()


    def _load_meta(g):
        blk = sid * _BLOCKS_PER_SUB + g * _GRP
        ms = g % 4
        pltpu.async_copy(meta_h.at[pl.ds(blk, _GRP)], meta_v.at[ms],
                         isem.at[ms])

    def _fire_gathers(slot, g):
        ms = g % 4
        pltpu.make_async_copy(meta_h.at[pl.ds(0, _GRP)], meta_v.at[ms],
                              isem.at[ms]).wait()
        for jj in range(_GRP):
            pltpu.async_copy(feat2_h.at[meta_v.at[ms, jj, cid + 1]],
                             rowbuf.at[slot, pl.ds(jj * _B, _B)],
                             gsem.at[slot])

    _load_meta(0)
    _fire_gathers(0, 0)
    _load_meta(1)

    def group_body(g, _):
        b = g % 2
        ms = g % 4
        pltpu.make_async_copy(feat2_h.at[pl.ds(0, _GB)], rowbuf.at[b],
                              gsem.at[b]).wait()

        @pl.when(g >= 1)
        def _():
            pltpu.make_async_copy(feat2_h.at[pl.ds(0, _GB)],
                                  rowbuf.at[1 - b], ssem.at[1 - b]).wait()

        @pl.when(g < _GROUPS - 1)
        def _():
            _fire_gathers(1 - b, g + 1)

        @pl.when(g < _GROUPS - 2)
        def _():
            _load_meta(g + 2)

        for jj in range(_GRP):
            @plsc.parallel_loop(0, _B, unroll=4)
            def _(e):
                se = jnp.full((16,), e, jnp.int32)
                v = plsc.bitcast(
                    plsc.load_gather(meta_v.at[ms, jj, 3], [se]), jnp.float32)
                e2 = e + jj * _B
                a = rowbuf[b, e2, pl.ds(0, 16)]
                c = rowbuf[b, e2, pl.ds(16, 16)]
                rowbuf[b, e2, pl.ds(0, 16)] = a * v
                rowbuf[b, e2, pl.ds(16, 16)] = c * v

        for jj in range(_GRP):
            pltpu.async_copy(rowbuf.at[b, pl.ds(jj * _B, _B)],
                             acc.at[meta_v.at[ms, jj, 0]], ssem.at[b],
                             add=True)
        return 0

    lax.fori_loop(0, _GROUPS, group_body, 0)
    last = (_GROUPS - 1) % 2
    pltpu.make_async_copy(feat2_h.at[pl.ds(0, _GB)], rowbuf.at[last],
                          ssem.at[last]).wait()
    plsc.subcore_barrier()

    @pl.when(cid == 0)
    def _():
        pltpu.sync_copy(acc.at[pl.ds(r0, _ROWS_PER_SUB)],
                        out_lo.at[pl.ds(r0, _ROWS_PER_SUB)])

    @pl.when(cid == 1)
    def _():
        pltpu.sync_copy(acc.at[pl.ds(r0, _ROWS_PER_SUB)],
                        out_hi.at[pl.ds(r0, _ROWS_PER_SUB)])


@jax.jit
def _spmm(feat2, meta):
    mesh = plsc.VectorSubcoreMesh(core_axis_name="c", subcore_axis_name="s")
    f = pl.kernel(
        _spmm_kernel,
        out_type=(jax.ShapeDtypeStruct((_N_PAD, 32), jnp.float32),
                  jax.ShapeDtypeStruct((_N_PAD, 32), jnp.float32)),
        mesh=mesh,
        compiler_params=pltpu.CompilerParams(needs_layout_passes=False,
                                             use_tc_tiling_on_sc=False),
        scratch_types=[
            pltpu.VMEM_SHARED((_N_PAD, 32), jnp.float32),
            pltpu.VMEM((4, _GRP, 4, _B), jnp.int32),
            pltpu.VMEM((2, _GRP * _B, 32), jnp.float32),
            pltpu.SemaphoreType.DMA((2,)),
            pltpu.SemaphoreType.DMA((2,)),
            pltpu.SemaphoreType.DMA((4,)),
        ],
    )
    return f(feat2, meta)


def _dense_body(f_ref, t1lo_ref, t1hi_ref, w_ref, b_ref, o_ref):
    f = f_ref[...]
    t1 = jnp.concatenate([t1lo_ref[...], t1hi_ref[...]], axis=1)
    t2 = t1 * f
    dn = (((1,), (1,)), ((), ()))
    z = (lax.dot_general(t1, w_ref[0], dn, precision=lax.Precision.DEFAULT,
                         preferred_element_type=jnp.float32)
         + lax.dot_general(t2, w_ref[1], dn, precision=lax.Precision.DEFAULT,
                           preferred_element_type=jnp.float32)
         + lax.dot_general(f, w_ref[2], dn, precision=lax.Precision.DEFAULT,
                           preferred_element_type=jnp.float32)
         + b_ref[...])
    z = jnp.where(z >= 0, z, 0.01 * z)
    s = jnp.sum(z * z, axis=1, keepdims=True)
    o_ref[...] = z * lax.rsqrt(jnp.maximum(s, 1e-24))


def _dense(feature, t1lo, t1hi, W, bsum):
    BN = 2000
    nblk = _N // BN
    return pl.pallas_call(
        _dense_body,
        grid=(nblk,),
        in_specs=[
            pl.BlockSpec((BN, _D), lambda i: (i, 0)),
            pl.BlockSpec((BN, 32), lambda i: (i, 0)),
            pl.BlockSpec((BN, 32), lambda i: (i, 0)),
            pl.BlockSpec((3, _D, _D), lambda i: (0, 0, 0)),
            pl.BlockSpec((1, _D), lambda i: (0, 0)),
        ],
        out_specs=pl.BlockSpec((BN, _D), lambda i: (i, 0)),
        out_shape=jax.ShapeDtypeStruct((_N, _D), jnp.float32),
    )(feature, t1lo, t1hi, W, bsum)


def kernel(l_indices, l_values, user_emb, item_emb, Ws, bs):
    rows = l_indices[0].astype(jnp.int32)
    cols = l_indices[1].astype(jnp.int32)
    pad = _E_PAD - _E
    spread = (jnp.arange(pad, dtype=jnp.int32) * 17) % _N
    rows_p = jnp.concatenate([rows, spread])
    cols_p = jnp.concatenate([cols, spread])
    vals_p = jnp.concatenate([l_values, jnp.zeros((pad,), jnp.float32)])
    meta = jnp.stack([
        rows_p.reshape(_NB, _B),
        (cols_p * 2).reshape(_NB, _B),
        (cols_p * 2 + 1).reshape(_NB, _B),
        lax.bitcast_convert_type(vals_p, jnp.int32).reshape(_NB, _B),
    ], axis=1)

    feature = jnp.concatenate([user_emb, item_emb], axis=0)
    all_embs = [feature]
    for i in range(_L):
        t1lo, t1hi = _spmm(feature.reshape(2 * _N, 32), meta)
        feature = _dense(feature, t1lo, t1hi, Ws[i], bs[i].sum(0)[None, :])
        all_embs.append(feature)
    all_e = jnp.concatenate(all_embs, axis=1)
    return all_e[:_N_USERS], all_e[_N_USERS:]
